# baseline (speedup 1.0000x reference)
import functools

import jax
import jax.numpy as jnp
from jax import lax
from jax.experimental import pallas as pl
from jax.experimental.pallas import tpu as pltpu
from jax.experimental.pallas import tpu_sc as plsc

N = 16384
C = 1000
K = N // 2

SROWS = 4096
TROWS = N - SROWS
NW = 32
RPW = SROWS // NW
CHR = 16
NCHUNK = RPW // CHR

CH = 1024
NCH = TROWS // CH
NBUF = 4

_SC_MESH = plsc.VectorSubcoreMesh(core_axis_name="c", subcore_axis_name="s")


@functools.partial(
    pl.kernel,
    mesh=_SC_MESH,
    out_type=[
        jax.ShapeDtypeStruct((SROWS,), jnp.float32),
        jax.ShapeDtypeStruct((SROWS,), jnp.float32),
        jax.ShapeDtypeStruct((SROWS,), jnp.float32),
    ],
    scratch_types=[
        pltpu.VMEM((CHR, C), jnp.float32),
        pltpu.VMEM((RPW,), jnp.int32),
        pltpu.VMEM((RPW,), jnp.float32),
        pltpu.VMEM((RPW,), jnp.float32),
        pltpu.VMEM((RPW,), jnp.float32),
    ],
    compiler_params=pltpu.CompilerParams(use_tc_tiling_on_sc=False, needs_layout_passes=False),
)
def _sc_rows(pred_hbm, tgt_hbm, m_hbm, s_hbm, p_hbm,
             xbuf, tbuf, mbuf, sbuf, pbuf):
    wid = lax.axis_index("s") * 2 + lax.axis_index("c")
    base = wid * RPW
    pltpu.sync_copy(tgt_hbm.at[pl.ds(base, RPW)], tbuf)
    lanes = lax.iota(jnp.int32, 16)

    for cc in range(NCHUNK):
        row0 = TROWS + base + cc * CHR
        pltpu.sync_copy(pred_hbm.at[pl.ds(row0, CHR), :], xbuf)
        tvec = tbuf[pl.ds(cc * CHR, 16)]

        def row_body(r, carry):
            mvec, svec, pvec = carry
            v = xbuf[r, pl.ds(0, 16)]
            mx = v
            for j in range(1, 62):
                mx = jnp.maximum(mx, xbuf[r, pl.ds(j * 16, 16)])
            vt = xbuf[r, pl.ds(C - 16, 16)]
            mx = jnp.maximum(mx, vt)
            m_row = jnp.max(mx)
            acc = jnp.exp(v - m_row)
            for j in range(1, 62):
                acc = acc + jnp.exp(xbuf[r, pl.ds(j * 16, 16)] - m_row)
            acc = acc + jnp.where(lanes >= 8, jnp.exp(vt - m_row), 0.0)
            s_row = jnp.sum(acc)
            t_r = jnp.sum(jnp.where(lanes == r, tvec, 0))
            off = pl.multiple_of((t_r // 16) * 16, 16)
            tv = xbuf[r, pl.ds(off, 16)]
            p_row = jnp.sum(jnp.where(lanes == t_r % 16, tv, 0.0))
            sel = lanes == r
            return (jnp.where(sel, m_row, mvec), jnp.where(sel, s_row, svec),
                    jnp.where(sel, p_row, pvec))

        mvec, svec, pvec = lax.fori_loop(
            0, CHR, row_body,
            (jnp.zeros((16,), jnp.float32), jnp.zeros((16,), jnp.float32),
             jnp.zeros((16,), jnp.float32)))
        mbuf[pl.ds(cc * CHR, 16)] = mvec
        sbuf[pl.ds(cc * CHR, 16)] = svec
        pbuf[pl.ds(cc * CHR, 16)] = pvec

    pltpu.sync_copy(mbuf, m_hbm.at[pl.ds(base, RPW)])
    pltpu.sync_copy(sbuf, s_hbm.at[pl.ds(base, RPW)])
    pltpu.sync_copy(pbuf, p_hbm.at[pl.ds(base, RPW)])


def _tc_body(pred_hbm, tgt_ref, loss_ref, buf, sems):
    def start(i):
        pltpu.make_async_copy(
            pred_hbm.at[pl.ds(i * CH, CH), :], buf.at[i % NBUF],
            sems.at[i % NBUF]).start()

    for i in range(NBUF):
        start(i)

    for c in range(NCH):
        pltpu.make_async_copy(
            pred_hbm.at[pl.ds(c * CH, CH), :], buf.at[c % NBUF],
            sems.at[c % NBUF]).wait()
        x = buf[c % NBUF]
        t = tgt_ref[c, 0, :]
        m = jnp.max(x, axis=1, keepdims=True)
        s = jnp.sum(jnp.exp(x - m), axis=1, keepdims=True)
        lse = m + jnp.log(s)
        col = jax.lax.broadcasted_iota(jnp.int32, x.shape, 1)
        picked = jnp.sum(jnp.where(col == t[:, None], x, 0.0), axis=1,
                         keepdims=True)
        loss_ref[c, :] = (lse - picked)[:, 0]
        if c + NBUF < NCH:
            start(c + NBUF)


def _select_body(loss_ref, m_ref, s_ref, p_ref, out_ref):
    loss_sc = m_ref[...] + jnp.log(s_ref[...]) - p_ref[...]
    x1 = loss_ref[...]
    b1 = jax.lax.bitcast_convert_type(x1, jnp.int32)
    b2 = jax.lax.bitcast_convert_type(loss_sc, jnp.int32)

    def bsearch(_, carry):
        lo, hi = carry
        mid = lo + (hi - lo) // 2
        cnt = (jnp.sum((b1 >= mid).astype(jnp.int32))
               + jnp.sum((b2 >= mid).astype(jnp.int32)))
        take = cnt >= K
        return jnp.where(take, mid, lo), jnp.where(take, hi, mid)

    lo, _ = jax.lax.fori_loop(0, 31, bsearch,
                              (jnp.int32(0), jnp.int32(0x7F800000)))
    thr = jax.lax.bitcast_convert_type(lo, jnp.float32)
    g1, g2 = b1 > lo, b2 > lo
    cnt_gt = (jnp.sum(g1.astype(jnp.int32)) + jnp.sum(g2.astype(jnp.int32)))
    sum_gt = (jnp.sum(jnp.where(g1, x1, 0.0))
              + jnp.sum(jnp.where(g2, loss_sc, 0.0)))
    total = sum_gt + (K - cnt_gt).astype(jnp.float32) * thr
    out_ref[...] = jnp.reshape(total / jnp.float32(K), (1, 1))


@jax.jit
def kernel(pred, target):
    tgt_tc = target[:TROWS].reshape(NCH, 1, CH)
    tgt_sc = target[TROWS:]

    m_sc, s_sc, p_sc = _sc_rows(pred, tgt_sc)

    loss_tc = pl.pallas_call(
        _tc_body,
        in_specs=[
            pl.BlockSpec(memory_space=pltpu.MemorySpace.HBM),
            pl.BlockSpec(memory_space=pltpu.MemorySpace.VMEM),
        ],
        out_specs=pl.BlockSpec(memory_space=pltpu.MemorySpace.VMEM),
        out_shape=jax.ShapeDtypeStruct((NCH, CH), jnp.float32),
        scratch_shapes=[
            pltpu.VMEM((NBUF, CH, C), jnp.float32),
            pltpu.SemaphoreType.DMA((NBUF,)),
        ],
    )(pred, tgt_tc)

    out = pl.pallas_call(
        _select_body,
        out_shape=jax.ShapeDtypeStruct((1, 1), jnp.float32),
    )(loss_tc,
      m_sc.reshape(SROWS // CH, CH),
      s_sc.reshape(SROWS // CH, CH),
      p_sc.reshape(SROWS // CH, CH))
    return out[0, 0]


# speedup vs baseline: 1.2872x; 1.2872x over previous
import functools

import jax
import jax.numpy as jnp
from jax import lax
from jax.experimental import pallas as pl
from jax.experimental.pallas import tpu as pltpu
from jax.experimental.pallas import tpu_sc as plsc

N = 16384
C = 1000
K = N // 2

SROWS = 4096
TROWS = N - SROWS
NW = 32
RPW = SROWS // NW
CHR = 16
NCHUNK = RPW // CHR

CH = 1024
NCH = TROWS // CH
NBUF = 4

_SC_MESH = plsc.VectorSubcoreMesh(core_axis_name="c", subcore_axis_name="s")


@functools.partial(
    pl.kernel,
    mesh=_SC_MESH,
    out_type=[
        jax.ShapeDtypeStruct((SROWS,), jnp.float32),
        jax.ShapeDtypeStruct((SROWS,), jnp.float32),
        jax.ShapeDtypeStruct((SROWS,), jnp.float32),
    ],
    scratch_types=[
        pltpu.VMEM((CHR, C), jnp.float32),
        pltpu.VMEM((RPW,), jnp.int32),
        pltpu.VMEM((RPW,), jnp.float32),
        pltpu.VMEM((RPW,), jnp.float32),
        pltpu.VMEM((RPW,), jnp.float32),
    ],
    compiler_params=pltpu.CompilerParams(use_tc_tiling_on_sc=False, needs_layout_passes=False),
)
def _sc_rows(pred_hbm, tgt_hbm, m_hbm, s_hbm, p_hbm,
             xbuf, tbuf, mbuf, sbuf, pbuf):
    wid = lax.axis_index("s") * 2 + lax.axis_index("c")
    base = wid * RPW
    pltpu.sync_copy(tgt_hbm.at[pl.ds(base, RPW)], tbuf)
    lanes = lax.iota(jnp.int32, 16)

    for cc in range(NCHUNK):
        row0 = base + cc * CHR
        pltpu.sync_copy(pred_hbm.at[pl.ds(row0, CHR), :], xbuf)
        tvec = tbuf[pl.ds(cc * CHR, 16)]

        def row_body(r, carry):
            mvec, svec, pvec = carry
            v = xbuf[r, pl.ds(0, 16)]
            mx = v
            for j in range(1, 62):
                mx = jnp.maximum(mx, xbuf[r, pl.ds(j * 16, 16)])
            vt = xbuf[r, pl.ds(C - 16, 16)]
            mx = jnp.maximum(mx, vt)
            m_row = jnp.max(mx)
            acc = jnp.exp(v - m_row)
            for j in range(1, 62):
                acc = acc + jnp.exp(xbuf[r, pl.ds(j * 16, 16)] - m_row)
            acc = acc + jnp.where(lanes >= 8, jnp.exp(vt - m_row), 0.0)
            s_row = jnp.sum(acc)
            t_r = jnp.sum(jnp.where(lanes == r, tvec, 0))
            off = pl.multiple_of((t_r // 16) * 16, 16)
            tv = xbuf[r, pl.ds(off, 16)]
            p_row = jnp.sum(jnp.where(lanes == t_r % 16, tv, 0.0))
            sel = lanes == r
            return (jnp.where(sel, m_row, mvec), jnp.where(sel, s_row, svec),
                    jnp.where(sel, p_row, pvec))

        mvec, svec, pvec = lax.fori_loop(
            0, CHR, row_body,
            (jnp.zeros((16,), jnp.float32), jnp.zeros((16,), jnp.float32),
             jnp.zeros((16,), jnp.float32)))
        mbuf[pl.ds(cc * CHR, 16)] = mvec
        sbuf[pl.ds(cc * CHR, 16)] = svec
        pbuf[pl.ds(cc * CHR, 16)] = pvec

    pltpu.sync_copy(mbuf, m_hbm.at[pl.ds(base, RPW)])
    pltpu.sync_copy(sbuf, s_hbm.at[pl.ds(base, RPW)])
    pltpu.sync_copy(pbuf, p_hbm.at[pl.ds(base, RPW)])


def _tc_body(pred_hbm, tgt_ref, loss_ref, buf, sems):
    def start(i):
        pltpu.make_async_copy(
            pred_hbm.at[pl.ds(i * CH, CH), :], buf.at[i % NBUF],
            sems.at[i % NBUF]).start()

    for i in range(NBUF):
        start(i)

    for c in range(NCH):
        pltpu.make_async_copy(
            pred_hbm.at[pl.ds(c * CH, CH), :], buf.at[c % NBUF],
            sems.at[c % NBUF]).wait()
        x = buf[c % NBUF]
        t = tgt_ref[c, 0, :]
        m = jnp.max(x, axis=1, keepdims=True)
        s = jnp.sum(jnp.exp(x - m), axis=1, keepdims=True)
        lse = m + jnp.log(s)
        col = jax.lax.broadcasted_iota(jnp.int32, x.shape, 1)
        picked = jnp.sum(jnp.where(col == t[:, None], x, 0.0), axis=1,
                         keepdims=True)
        loss_ref[c, :] = (lse - picked)[:, 0]
        if c + NBUF < NCH:
            start(c + NBUF)


def _select_body(loss_ref, m_ref, s_ref, p_ref, out_ref):
    loss_sc = m_ref[...] + jnp.log(s_ref[...]) - p_ref[...]
    x1 = loss_ref[...]
    b1 = jax.lax.bitcast_convert_type(x1, jnp.int32)
    b2 = jax.lax.bitcast_convert_type(loss_sc, jnp.int32)

    def bsearch(_, carry):
        lo, hi = carry
        mid = lo + (hi - lo) // 2
        cnt = (jnp.sum((b1 >= mid).astype(jnp.int32))
               + jnp.sum((b2 >= mid).astype(jnp.int32)))
        take = cnt >= K
        return jnp.where(take, mid, lo), jnp.where(take, hi, mid)

    lo, _ = jax.lax.fori_loop(0, 31, bsearch,
                              (jnp.int32(0), jnp.int32(0x7F800000)))
    thr = jax.lax.bitcast_convert_type(lo, jnp.float32)
    g1, g2 = b1 > lo, b2 > lo
    cnt_gt = (jnp.sum(g1.astype(jnp.int32)) + jnp.sum(g2.astype(jnp.int32)))
    sum_gt = (jnp.sum(jnp.where(g1, x1, 0.0))
              + jnp.sum(jnp.where(g2, loss_sc, 0.0)))
    total = sum_gt + (K - cnt_gt).astype(jnp.float32) * thr
    out_ref[...] = jnp.reshape(total / jnp.float32(K), (1, 1))


@jax.jit
def kernel(pred, target):
    tgt_tc = target[:TROWS].reshape(NCH, 1, CH)
    tgt_sc = target[TROWS:]

    m_sc, s_sc, p_sc = _sc_rows(pred[TROWS:], tgt_sc)

    loss_tc = pl.pallas_call(
        _tc_body,
        in_specs=[
            pl.BlockSpec(memory_space=pltpu.MemorySpace.HBM),
            pl.BlockSpec(memory_space=pltpu.MemorySpace.VMEM),
        ],
        out_specs=pl.BlockSpec(memory_space=pltpu.MemorySpace.VMEM),
        out_shape=jax.ShapeDtypeStruct((NCH, CH), jnp.float32),
        scratch_shapes=[
            pltpu.VMEM((NBUF, CH, C), jnp.float32),
            pltpu.SemaphoreType.DMA((NBUF,)),
        ],
    )(pred, tgt_tc)

    out = pl.pallas_call(
        _select_body,
        out_shape=jax.ShapeDtypeStruct((1, 1), jnp.float32),
    )(loss_tc,
      m_sc.reshape(SROWS // CH, CH),
      s_sc.reshape(SROWS // CH, CH),
      p_sc.reshape(SROWS // CH, CH))
    return out[0, 0]


# speedup vs baseline: 2.1460x; 1.6673x over previous
import jax
import jax.numpy as jnp
from jax.experimental import pallas as pl
from jax.experimental.pallas import tpu as pltpu

N = 16384
C = 1000
K = N // 2
CH = 1024
NCH = N // CH
NBUF = 4


def _body(pred_hbm, tgt_ref, out_ref, buf, loss_ref, sems):
    def start(i):
        pltpu.make_async_copy(
            pred_hbm.at[pl.ds(i * CH, CH), :], buf.at[i % NBUF],
            sems.at[i % NBUF]).start()

    for i in range(NBUF):
        start(i)

    for c in range(NCH):
        pltpu.make_async_copy(
            pred_hbm.at[pl.ds(c * CH, CH), :], buf.at[c % NBUF],
            sems.at[c % NBUF]).wait()
        x = buf[c % NBUF]
        t = tgt_ref[c, 0, :]
        m = jnp.max(x, axis=1, keepdims=True)
        s = jnp.sum(jnp.exp(x - m), axis=1, keepdims=True)
        lse = m + jnp.log(s)
        col = jax.lax.broadcasted_iota(jnp.int32, x.shape, 1)
        picked = jnp.sum(jnp.where(col == t[:, None], x, 0.0), axis=1,
                         keepdims=True)
        loss_ref[c, :] = (lse - picked)[:, 0]
        if c + NBUF < NCH:
            start(c + NBUF)

    x = loss_ref[...]
    bits = jax.lax.bitcast_convert_type(x, jnp.int32)

    def bsearch(_, carry):
        lo, hi = carry
        mid = lo + (hi - lo) // 2
        cnt = jnp.sum((bits >= mid).astype(jnp.int32))
        take = cnt >= K
        return jnp.where(take, mid, lo), jnp.where(take, hi, mid)

    lo, _ = jax.lax.fori_loop(0, 31, bsearch,
                              (jnp.int32(0), jnp.int32(0x7F800000)))
    thr = jax.lax.bitcast_convert_type(lo, jnp.float32)
    gt = bits > lo
    cnt_gt = jnp.sum(gt.astype(jnp.int32))
    sum_gt = jnp.sum(jnp.where(gt, x, 0.0))
    total = sum_gt + (K - cnt_gt).astype(jnp.float32) * thr
    out_ref[...] = jnp.reshape(total / jnp.float32(K), (1, 1))


@jax.jit
def kernel(pred, target):
    tgt3 = target.reshape(NCH, 1, CH)
    out = pl.pallas_call(
        _body,
        in_specs=[
            pl.BlockSpec(memory_space=pltpu.MemorySpace.HBM),
            pl.BlockSpec(memory_space=pltpu.MemorySpace.VMEM),
        ],
        out_specs=pl.BlockSpec(memory_space=pltpu.MemorySpace.VMEM),
        out_shape=jax.ShapeDtypeStruct((1, 1), jnp.float32),
        scratch_shapes=[
            pltpu.VMEM((NBUF, CH, C), jnp.float32),
            pltpu.VMEM((NCH, CH), jnp.float32),
            pltpu.SemaphoreType.DMA((NBUF,)),
        ],
    )(pred, tgt3)
    return out[0, 0]
